# baseline (speedup 1.0000x reference)
import functools

import numpy as np
import jax
import jax.numpy as jnp
from jax import lax
from jax.experimental import pallas as pl
from jax.experimental.pallas import tpu as pltpu
from jax.experimental.pallas import tpu_sc as plsc

D_MODEL = 128
GLYPH_RES = 128
FIRST_SYSTEM_TOKEN = 16384
VOCAB_SIZE = 16448
SLABS = 130
VOCAB_PAD = SLABS * GLYPH_RES
FUSE_GRID = 5
SLABS_PER_BLK = SLABS // FUSE_GRID
NUM_CORES = 2
NUM_SUBCORES = 16
NUM_WORKERS = NUM_CORES * NUM_SUBCORES
CHUNK = 256
GCH = 128
NBUF = 3
LOOKAHEAD = 2


def _sine_pe(seq_len, d_model):
    pos = np.arange(seq_len)[:, None].astype(np.float32)
    div = np.exp(np.arange(0, d_model, 2).astype(np.float32)
                 * (-np.log(10000.0) / d_model))
    pe = np.zeros((seq_len, d_model), dtype=np.float32)
    pe[:, 0::2] = np.sin(pos * div)
    pe[:, 1::2] = np.cos(pos * div)
    return pe


def _fuse_body(tok_ref, cxs_ref, cys_ref, o_ref):
    k = pl.program_id(0)
    for j in range(SLABS_PER_BLK):
        slab = k * SLABS_PER_BLK + j
        regular = slab < GLYPH_RES
        yidx = jnp.where(regular, slab, 0)
        yrow = cys_ref[pl.ds(yidx, 1), :]
        xrows = jnp.where(regular, cxs_ref[...], cxs_ref[pl.ds(0, 1), :])
        lo = j * GLYPH_RES
        o_ref[lo:lo + GLYPH_RES, :] = (
            tok_ref[lo:lo + GLYPH_RES, :] + xrows + yrow)


def _build_fused(token_table, cxs, cys):
    blk = SLABS_PER_BLK * GLYPH_RES
    return pl.pallas_call(
        _fuse_body,
        grid=(FUSE_GRID,),
        in_specs=[
            pl.BlockSpec((blk, D_MODEL), lambda k: (k, 0)),
            pl.BlockSpec((GLYPH_RES, D_MODEL), lambda k: (0, 0)),
            pl.BlockSpec((GLYPH_RES, D_MODEL), lambda k: (0, 0)),
        ],
        out_specs=pl.BlockSpec((blk, D_MODEL), lambda k: (k, 0)),
        out_shape=jax.ShapeDtypeStruct((VOCAB_PAD, D_MODEL), jnp.float32),
    )(token_table, cxs, cys)


def _make_sc_gather(batch, seq_len):
    s_per_w = seq_len // NUM_WORKERS
    bchunks = batch // CHUNK
    nchunks = s_per_w * bchunks
    per_w = s_per_w * batch
    mesh = plsc.VectorSubcoreMesh(
        core_axis_name="c", subcore_axis_name="s",
        num_cores=NUM_CORES, num_subcores=NUM_SUBCORES)

    @functools.partial(
        pl.kernel,
        out_type=jax.ShapeDtypeStruct((batch, seq_len, D_MODEL), jnp.float32),
        mesh=mesh,
        scratch_types=(
            [pltpu.VMEM((per_w,), jnp.int32)]
            + [pltpu.VMEM((CHUNK, D_MODEL), jnp.float32)] * NBUF
            + [pltpu.VMEM((s_per_w, D_MODEL), jnp.float32)]
            + [pltpu.SemaphoreType.DMA] * (2 * NBUF)
        ),
    )
    def sc_gather(tokt_hbm, fused_hbm, pe_hbm, out_hbm,
                  idx_all, rows0, rows1, rows2, pe_v,
                  gsem0, gsem1, gsem2, osem0, osem1, osem2):
        rows = (rows0, rows1, rows2)
        gsem = (gsem0, gsem1, gsem2)
        osem = (osem0, osem1, osem2)
        wid = lax.axis_index("s") * NUM_CORES + lax.axis_index("c")
        s_base = wid * s_per_w
        pltpu.sync_copy(pe_hbm.at[pl.ds(s_base, s_per_w)], pe_v)
        pltpu.sync_copy(tokt_hbm.at[pl.ds(s_base * batch, per_w)], idx_all)

        def gather_wait(p):
            for h in range(CHUNK // GCH):
                pltpu.make_async_copy(
                    fused_hbm.at[idx_all.at[pl.ds(0, GCH)]],
                    rows[p].at[pl.ds(h * GCH, GCH)], gsem[p]).wait()

        def out_wait(p):
            pltpu.make_async_copy(
                rows[p], out_hbm.at[pl.ds(0, CHUNK), 0], osem[p]).wait()

        def start(m, p):
            for h in range(CHUNK // GCH):
                pltpu.async_copy(
                    fused_hbm.at[idx_all.at[pl.ds(m * CHUNK + h * GCH, GCH)]],
                    rows[p].at[pl.ds(h * GCH, GCH)], gsem[p])

        def add_and_out(m, p):
            s_off = m // bchunks
            b0 = lax.rem(m, bchunks) * CHUNK
            pe_regs = [pe_v[s_off, pl.ds(16 * c, 16)]
                       for c in range(D_MODEL // 16)]

            @plsc.parallel_loop(0, CHUNK, unroll=8)
            def _add_pe(r):
                for c in range(D_MODEL // 16):
                    sl = pl.ds(16 * c, 16)
                    rows[p][r, sl] = rows[p][r, sl] + pe_regs[c]

            pltpu.async_copy(rows[p],
                             out_hbm.at[pl.ds(b0, CHUNK), s_base + s_off],
                             osem[p])

        for m0 in range(LOOKAHEAD):
            start(m0, m0)

        def step(m, b):
            p = b % NBUF
            rb = (b + LOOKAHEAD) % NBUF

            @pl.when(m + LOOKAHEAD < nchunks)
            def _refill():
                @pl.when(m >= NBUF - LOOKAHEAD)
                def _():
                    out_wait(rb)
                start(m + LOOKAHEAD, rb)

            gather_wait(p)
            add_and_out(m, p)

        def body(i, carry):
            for b in range(NBUF):
                step(NBUF * i + b, b)
            return carry

        whole = (nchunks // NBUF) * NBUF
        lax.fori_loop(0, nchunks // NBUF, body, 0)
        for m in range(whole, nchunks):
            gather_wait(m % NBUF)
            add_and_out(m, m % NBUF)
        for p in range(NBUF):
            out_wait(p)

    return sc_gather


def kernel(font_tokens, token_table, coord_x_table, coord_y_table):
    batch, seq_len = font_tokens.shape

    cxs = coord_x_table[1:GLYPH_RES + 1]
    cys = coord_y_table[1:GLYPH_RES + 1]
    fused = _build_fused(token_table, cxs, cys)

    pe = jnp.asarray(_sine_pe(seq_len, D_MODEL))
    tokt = font_tokens.T.reshape(-1)
    sc_gather = _make_sc_gather(batch, seq_len)
    return sc_gather(tokt, fused, pe)
